# Initial kernel scaffold; baseline (speedup 1.0000x reference)
import jax
import jax.numpy as jnp
from jax.experimental import pallas as pl


def kernel(state, action, expert_states, expert_actions):
    raise NotImplementedError("write your pallas kernel here")


# speedup vs baseline: 1.4609x; 1.4609x over previous
import functools
from math import sqrt

import jax
import jax.numpy as jnp
from jax.experimental import pallas as pl
from jax.experimental.pallas import tpu as pltpu

TIME_HORIZON = 1000
REWARD_SCALE = 5.0
REWARD_BANDWIDTH_SCALE = 5.0


def _disc_kernel(state_ref, action_ref, es_ref, ea_ref, out_ref,
                 sum_s, sumsq_s, sum_a, sumsq_a, inv_s, inv_a, dist,
                 *, k_total, take_n, weight, expert_w, bandwidth):
    p = pl.program_id(0)
    i = pl.program_id(1)
    nb = pl.num_programs(1)

    @pl.when(jnp.logical_and(p == 0, i == 0))
    def _init():
        sum_s[...] = jnp.zeros_like(sum_s)
        sumsq_s[...] = jnp.zeros_like(sumsq_s)
        sum_a[...] = jnp.zeros_like(sum_a)
        sumsq_a[...] = jnp.zeros_like(sumsq_a)

    @pl.when(p == 0)
    def _stats():
        es = es_ref[...]
        ea = ea_ref[...]
        sum_s[...] += jnp.sum(es, axis=0, keepdims=True)
        sumsq_s[...] += jnp.sum(es * es, axis=0, keepdims=True)
        sum_a[...] += jnp.sum(ea, axis=0, keepdims=True)
        sumsq_a[...] += jnp.sum(ea * ea, axis=0, keepdims=True)

    @pl.when(jnp.logical_and(p == 0, i == nb - 1))
    def _finalize_stats():
        kf = jnp.float32(k_total)
        mean_s = sum_s[...] / kf
        var_s = jnp.maximum(sumsq_s[...] / kf - mean_s * mean_s, 0.0)
        inv_s[...] = 1.0 / (jnp.sqrt(var_s) + 1e-8)
        mean_a = sum_a[...] / kf
        var_a = jnp.maximum(sumsq_a[...] / kf - mean_a * mean_a, 0.0)
        inv_a[...] = 1.0 / (jnp.sqrt(var_a) + 1e-8)

    @pl.when(p == 1)
    def _dists():
        zs = (es_ref[...] - state_ref[...]) * inv_s[...]
        za = (ea_ref[...] - action_ref[...]) * inv_a[...]
        d2 = jnp.sum(zs * zs, axis=1) + jnp.sum(za * za, axis=1)
        dist[i, :] = jnp.sqrt(d2)

    @pl.when(jnp.logical_and(p == 1, i == nb - 1))
    def _select():
        d = dist[...]
        bits = jax.lax.bitcast_convert_type(d, jnp.int32)

        def body(_, carry):
            lo, hi = carry
            mid = lo + (hi - lo) // 2
            cnt = jnp.sum((bits <= mid).astype(jnp.int32))
            ok = cnt >= take_n
            return (jnp.where(ok, lo, mid + 1), jnp.where(ok, mid, hi))

        lo, _ = jax.lax.fori_loop(
            0, 31, body, (jnp.int32(0), jnp.int32(0x7F800000)))
        val = jax.lax.bitcast_convert_type(lo, jnp.float32)
        less = bits < lo
        n_less = jnp.sum(less.astype(jnp.float32))
        s_less = jnp.sum(jnp.where(less, d, 0.0))
        cost = expert_w * s_less + (weight - n_less * expert_w) * val
        reward = REWARD_SCALE * jnp.exp(-bandwidth * cost)
        out_ref[...] = reward.reshape(1, 1)


def kernel(state, action, expert_states, expert_actions):
    k_total, state_size = expert_states.shape
    action_size = expert_actions.shape[1]
    br = 2000
    assert k_total % br == 0
    nb = k_total // br

    weight = 1.0 / TIME_HORIZON - 1e-6
    expert_w = 1.0 / k_total
    take_n = int(-(-weight // expert_w))
    d_atom = state_size + action_size
    bandwidth = REWARD_BANDWIDTH_SCALE * TIME_HORIZON / sqrt(d_atom)

    body = functools.partial(
        _disc_kernel, k_total=k_total, take_n=take_n, weight=weight,
        expert_w=expert_w, bandwidth=bandwidth)

    out = pl.pallas_call(
        body,
        grid=(2, nb),
        in_specs=[
            pl.BlockSpec((1, state_size), lambda p, i: (0, 0)),
            pl.BlockSpec((1, action_size), lambda p, i: (0, 0)),
            pl.BlockSpec((br, state_size), lambda p, i: (i, 0)),
            pl.BlockSpec((br, action_size), lambda p, i: (i, 0)),
        ],
        out_specs=pl.BlockSpec((1, 1), lambda p, i: (0, 0)),
        out_shape=jax.ShapeDtypeStruct((1, 1), jnp.float32),
        scratch_shapes=[
            pltpu.VMEM((1, state_size), jnp.float32),
            pltpu.VMEM((1, state_size), jnp.float32),
            pltpu.VMEM((1, action_size), jnp.float32),
            pltpu.VMEM((1, action_size), jnp.float32),
            pltpu.VMEM((1, state_size), jnp.float32),
            pltpu.VMEM((1, action_size), jnp.float32),
            pltpu.VMEM((nb, br), jnp.float32),
        ],
    )(state, action, expert_states, expert_actions)
    return out[0, 0]


# speedup vs baseline: 1.5812x; 1.0823x over previous
import functools
from math import sqrt

import jax
import jax.numpy as jnp
from jax.experimental import pallas as pl
from jax.experimental.pallas import tpu as pltpu

TIME_HORIZON = 1000
REWARD_SCALE = 5.0
REWARD_BANDWIDTH_SCALE = 5.0

_DIMS_NT = (((1,), (0,)), ((), ()))
_DIMS_TT = (((1,), (1,)), ((), ()))


def _disc_kernel(state_ref, action_ref, es_ref, ea_ref, out_ref,
                 sum_s, sumsq_s, sum_a, sumsq_a, ws, vs, wa, va, c_ref,
                 dist, *, k_total, take_n, weight, expert_w, bandwidth):
    p = pl.program_id(0)
    i = pl.program_id(1)
    nb = pl.num_programs(1)
    br = es_ref.shape[0]

    @pl.when(jnp.logical_and(p == 0, i == 0))
    def _init():
        sum_s[...] = jnp.zeros_like(sum_s)
        sumsq_s[...] = jnp.zeros_like(sumsq_s)
        sum_a[...] = jnp.zeros_like(sum_a)
        sumsq_a[...] = jnp.zeros_like(sumsq_a)

    @pl.when(p == 0)
    def _stats():
        es = es_ref[...]
        ea = ea_ref[...]
        ones = jnp.ones((1, br), jnp.float32)
        dot = functools.partial(
            jax.lax.dot_general, dimension_numbers=_DIMS_NT,
            preferred_element_type=jnp.float32)
        sum_s[...] += dot(ones, es)
        sumsq_s[...] += dot(ones, es * es)
        sum_a[...] += dot(ones, ea)
        sumsq_a[...] += dot(ones, ea * ea)

    @pl.when(jnp.logical_and(p == 0, i == nb - 1))
    def _finalize_stats():
        kf = jnp.float32(k_total)
        mean_s = sum_s[...] / kf
        var_s = jnp.maximum(sumsq_s[...] / kf - mean_s * mean_s, 0.0)
        inv_s = 1.0 / (jnp.sqrt(var_s) + 1e-8)
        w_s = inv_s * inv_s
        g_s = state_ref[...]
        ws[...] = w_s
        vs[...] = -2.0 * w_s * g_s
        mean_a = sum_a[...] / kf
        var_a = jnp.maximum(sumsq_a[...] / kf - mean_a * mean_a, 0.0)
        inv_a = 1.0 / (jnp.sqrt(var_a) + 1e-8)
        w_a = inv_a * inv_a
        g_a = action_ref[...]
        wa[...] = w_a
        va[...] = -2.0 * w_a * g_a
        c_ref[0, 0] = (jnp.sum(w_s * g_s * g_s) + jnp.sum(w_a * g_a * g_a))

    @pl.when(p == 1)
    def _dists():
        es = es_ref[...]
        ea = ea_ref[...]
        dot = functools.partial(
            jax.lax.dot_general, dimension_numbers=_DIMS_TT,
            preferred_element_type=jnp.float32)
        d2 = (dot(ws[...], es * es) + dot(vs[...], es)
              + dot(wa[...], ea * ea) + dot(va[...], ea))
        dist[i, :] = d2[0, :]

    @pl.when(jnp.logical_and(p == 1, i == nb - 1))
    def _select():
        d = jnp.sqrt(jnp.maximum(dist[...] + c_ref[0, 0], 0.0))
        bits = jax.lax.bitcast_convert_type(d, jnp.int32)

        def body(_, carry):
            lo, hi = carry
            mid = lo + (hi - lo) // 2
            cnt = jnp.sum((bits <= mid).astype(jnp.int32))
            ok = cnt >= take_n
            return (jnp.where(ok, lo, mid + 1), jnp.where(ok, mid, hi))

        lo, _ = jax.lax.fori_loop(
            0, 31, body, (jnp.int32(0), jnp.int32(0x7F800000)))
        val = jax.lax.bitcast_convert_type(lo, jnp.float32)
        less = bits < lo
        n_less = jnp.sum(less.astype(jnp.float32))
        s_less = jnp.sum(jnp.where(less, d, 0.0))
        cost = expert_w * s_less + (weight - n_less * expert_w) * val
        reward = REWARD_SCALE * jnp.exp(-bandwidth * cost)
        out_ref[...] = reward.reshape(1, 1)


def kernel(state, action, expert_states, expert_actions):
    k_total, state_size = expert_states.shape
    action_size = expert_actions.shape[1]
    br = 2000
    assert k_total % br == 0
    nb = k_total // br

    weight = 1.0 / TIME_HORIZON - 1e-6
    expert_w = 1.0 / k_total
    take_n = int(-(-weight // expert_w))
    d_atom = state_size + action_size
    bandwidth = REWARD_BANDWIDTH_SCALE * TIME_HORIZON / sqrt(d_atom)

    body = functools.partial(
        _disc_kernel, k_total=k_total, take_n=take_n, weight=weight,
        expert_w=expert_w, bandwidth=bandwidth)

    out = pl.pallas_call(
        body,
        grid=(2, nb),
        in_specs=[
            pl.BlockSpec((1, state_size), lambda p, i: (0, 0)),
            pl.BlockSpec((1, action_size), lambda p, i: (0, 0)),
            pl.BlockSpec((br, state_size), lambda p, i: (i, 0)),
            pl.BlockSpec((br, action_size), lambda p, i: (i, 0)),
        ],
        out_specs=pl.BlockSpec((1, 1), lambda p, i: (0, 0)),
        out_shape=jax.ShapeDtypeStruct((1, 1), jnp.float32),
        scratch_shapes=[
            pltpu.VMEM((1, state_size), jnp.float32),
            pltpu.VMEM((1, state_size), jnp.float32),
            pltpu.VMEM((1, action_size), jnp.float32),
            pltpu.VMEM((1, action_size), jnp.float32),
            pltpu.VMEM((1, state_size), jnp.float32),
            pltpu.VMEM((1, state_size), jnp.float32),
            pltpu.VMEM((1, action_size), jnp.float32),
            pltpu.VMEM((1, action_size), jnp.float32),
            pltpu.SMEM((1, 1), jnp.float32),
            pltpu.VMEM((nb, br), jnp.float32),
        ],
    )(state, action, expert_states, expert_actions)
    return out[0, 0]
